# baseline (speedup 1.0000x reference)
import functools

import jax
import jax.numpy as jnp
from jax.experimental import pallas as pl
from jax.experimental.pallas import tpu as pltpu
from jax.experimental.pallas import tpu_sc as plsc

_N = 83
_NP = 96
_E = 3403
_B = 32
_T = 50
_CIN = 32
_CH = 16
_COUT = 32
_P = 12
_EPS = 1e-5
_T1 = _T - 2
_T2 = _T - 4
_BN_CNT = _B * _T2 * _COUT


_EP = 3584
_EPT = _EP // 16
_CHUNKS = _EPT // 16


_STRIPE = (_NP * _NP) // 8
_DSLOT = 128


def _lhat_sc_kernel(src_hbm, dst_hbm, out_hbm, src_v, dst_v, degl, degbuf,
                    dinvl, lhatl, accv, tmpv, deg_all, lhat_all):
    cid = jax.lax.axis_index("c")
    sid = jax.lax.axis_index("s")

    @pl.when(cid == 0)
    def _core0():
        base = sid * _EPT
        pltpu.sync_copy(src_hbm.at[pl.ds(base, _EPT)], src_v)
        pltpu.sync_copy(dst_hbm.at[pl.ds(base, _EPT)], dst_v)
        zeros16 = jnp.zeros((16,), jnp.float32)
        lane = jax.lax.iota(jnp.int32, 16)
        for i in range(_DSLOT // 16):
            degl[pl.ds(i * 16, 16)] = zeros16

        for c in range(_CHUNKS):
            sv = src_v[pl.ds(c * 16, 16)]
            dv = dst_v[pl.ds(c * 16, 16)]
            w = jnp.where(sv != dv, 1.0, 0.0).astype(jnp.float32)
            for l in range(16):
                plsc.addupdate_scatter(degl, [sv], w, mask=lane == l)
        pltpu.sync_copy(degl, deg_all.at[pl.ds(sid * _DSLOT, _DSLOT)])
        plsc.subcore_barrier()

        pltpu.sync_copy(deg_all, degbuf)
        for i in range(6):
            d = degbuf[pl.ds(i * 16, 16)]
            for s2 in range(1, 16):
                d = d + degbuf[pl.ds(s2 * _DSLOT + i * 16, 16)]
            s = 0.25 * d + 1.0
            for _ in range(12):
                s = 0.5 * (s + d / s)
            dinvl[pl.ds(i * 16, 16)] = jnp.where(d > 0, 1.0 / s, 0.0)

        for i in range(_NP * _NP // 16):
            lhatl[pl.ds(i * 16, 16)] = zeros16

        for c in range(_CHUNKS):
            sv = src_v[pl.ds(c * 16, 16)]
            dv = dst_v[pl.ds(c * 16, 16)]
            ds_ = plsc.load_gather(dinvl, [sv])
            dd = plsc.load_gather(dinvl, [dv])
            w = jnp.where(sv != dv, 1.0, 0.0).astype(jnp.float32)
            norm = -(ds_ * w * dd)
            fidx = sv * _NP + dv
            for l in range(16):
                plsc.addupdate_scatter(lhatl, [fidx], norm, mask=lane == l)
        nn = _NP * _NP
        pltpu.sync_copy(lhatl, lhat_all.at[pl.ds(sid * nn, nn)])
        plsc.subcore_barrier()

        @pl.when(sid < 8)
        def _():
            pltpu.sync_copy(lhat_all.at[pl.ds(sid * _STRIPE, _STRIPE)], accv)
            for s2 in range(1, 16):
                pltpu.sync_copy(
                    lhat_all.at[pl.ds(s2 * nn + sid * _STRIPE, _STRIPE)], tmpv)
                for k in range(_STRIPE // 16):
                    accv[pl.ds(k * 16, 16)] = (accv[pl.ds(k * 16, 16)]
                                               + tmpv[pl.ds(k * 16, 16)])
            pltpu.sync_copy(accv, out_hbm.at[pl.ds(sid * _STRIPE, _STRIPE)])


def _lhat_call(ei_pad):
    mesh = plsc.VectorSubcoreMesh(core_axis_name="c", subcore_axis_name="s",
                                  num_cores=2, num_subcores=16)
    f32 = jnp.float32
    builder = functools.partial(
        pl.kernel,
        out_type=jax.ShapeDtypeStruct((_NP * _NP,), f32),
        mesh=mesh,
        compiler_params=pltpu.CompilerParams(needs_layout_passes=False),
        scratch_types=[
            pltpu.VMEM((_EPT,), jnp.int32),
            pltpu.VMEM((_EPT,), jnp.int32),
            pltpu.VMEM((_DSLOT,), f32),
            pltpu.VMEM((16 * _DSLOT,), f32),
            pltpu.VMEM((_NP,), f32),
            pltpu.VMEM((_NP * _NP,), f32),
            pltpu.VMEM((_STRIPE,), f32),
            pltpu.VMEM((_STRIPE,), f32),
            pltpu.VMEM_SHARED((16 * _DSLOT,), f32),
            pltpu.VMEM_SHARED((16 * _NP * _NP,), f32),
        ],
    )
    return builder(_lhat_sc_kernel)(ei_pad[0], ei_pad[1]).reshape(_NP, _NP)


def _main_kernel(x_ref, lhatT_ref, wp_ref, wq_ref, wr_ref, b1_ref,
                 cw0_ref, cw1_ref, cw2_ref, cb_ref,
                 vp_ref, vq_ref, vr_ref, b2_ref,
                 sum_ref, sumsq_ref, last_ref):
    x = x_ref[0]
    xf = x.reshape(_T * _NP, _CIN)
    x0 = xf[0:_T1 * _NP]
    x1 = xf[_NP:(_T1 + 1) * _NP]
    x2 = xf[2 * _NP:(_T1 + 2) * _NP]
    xc = jnp.concatenate([x0, x1, x2], axis=1)
    p = xc @ wp_ref[...] + b1_ref[0:1]
    q = xc @ wq_ref[...] + b1_ref[1:2]
    r = xc @ wr_ref[...] + b1_ref[2:3]
    f0 = jax.nn.relu(p * jax.nn.sigmoid(q) + r)

    lhatT = lhatT_ref[...]
    zc0 = f0.reshape(_T1, _NP, _CH).transpose(0, 2, 1).reshape(_T1 * _CH, _NP)
    a1 = jnp.dot(zc0, lhatT, preferred_element_type=jnp.float32)
    a2 = 2.0 * jnp.dot(a1, lhatT, preferred_element_type=jnp.float32) - zc0
    f1 = a1.reshape(_T1, _CH, _NP).transpose(0, 2, 1).reshape(_T1 * _NP, _CH)
    f2 = a2.reshape(_T1, _CH, _NP).transpose(0, 2, 1).reshape(_T1 * _NP, _CH)
    out = f0 @ cw0_ref[...] + f1 @ cw1_ref[...] + f2 @ cw2_ref[...] + cb_ref[...]
    tg = jax.nn.relu(out)

    y0 = tg[0:_T2 * _NP]
    y1 = tg[_NP:(_T2 + 1) * _NP]
    y2 = tg[2 * _NP:(_T2 + 2) * _NP]
    yc = jnp.concatenate([y0, y1, y2], axis=1)
    p2 = yc @ vp_ref[...] + b2_ref[0:1]
    q2 = yc @ vq_ref[...] + b2_ref[1:2]
    r2 = yc @ vr_ref[...] + b2_ref[2:3]
    h2 = jax.nn.relu(p2 * jax.nn.sigmoid(q2) + r2)
    h2_3 = h2.reshape(_T2, _NP, _COUT)

    s_tn = jnp.sum(h2_3, axis=2)
    ss_tn = jnp.sum(h2_3 * h2_3, axis=2)
    sum_ref[0] = jnp.sum(s_tn, axis=0, keepdims=True)
    sumsq_ref[0] = jnp.sum(ss_tn, axis=0, keepdims=True)
    last_ref[0] = h2_3[_T2 - 1]


def _final_kernel(sums_ref, sumsq_ref, last_ref, gamma_ref, beta_ref,
                  linw_ref, linb_ref, hide_ref, ts_ref, out_ref):
    s = jnp.sum(sums_ref[...], axis=1, keepdims=True)
    ss = jnp.sum(sumsq_ref[...], axis=1, keepdims=True)
    mean = s / _BN_CNT
    var = ss / _BN_CNT - mean * mean
    inv = gamma_ref[...] * jax.lax.rsqrt(jnp.abs(var) + _EPS)
    shift = beta_ref[...] - mean * inv
    last = last_ref[...]
    h = jax.nn.relu(last * inv[None] + shift[None])
    hl = h.reshape(_B * _NP, _COUT) @ linw_ref[...]
    pooled = jnp.sum(hl.reshape(_B, _NP, _P), axis=1) * (1.0 / _N) + linb_ref[...]
    out_ref[:, 0:_P] = pooled
    out_ref[:, _P:_P + 2] = hide_ref[...]
    out_ref[:, _P + 2:_P + 3] = ts_ref[...]


def kernel(agent_obs, hideout_obs, timestep_obs, num_agents, edge_index,
           tc1_w1, tc1_b1, tc1_w2, tc1_b2, tc1_w3, tc1_b3, cheb_w, cheb_b,
           tc2_w1, tc2_b1, tc2_w2, tc2_b2, tc2_w3, tc2_b3, bn_gamma, bn_beta,
           lin_w, lin_b):
    f32 = jnp.float32

    ei_pad = jnp.pad(edge_index, ((0, 0), (0, _EP - _E)))
    lhatT = _lhat_call(ei_pad)

    def cat_w(w):
        return jnp.concatenate([w[:, :, 0, k].T for k in range(3)], axis=0)

    wp, wq, wr = cat_w(tc1_w1), cat_w(tc1_w2), cat_w(tc1_w3)
    b1 = jnp.stack([tc1_b1, tc1_b2, tc1_b3])
    vp, vq, vr = cat_w(tc2_w1), cat_w(tc2_w2), cat_w(tc2_w3)
    b2 = jnp.stack([tc2_b1, tc2_b2, tc2_b3])
    cb = cheb_b.reshape(1, _CH)

    x_pad = jnp.pad(agent_obs, ((0, 0), (0, 0), (0, _NP - _N), (0, 0)))

    full = lambda shape: pl.BlockSpec(shape, lambda b: (0,) * len(shape))
    sums, sumsq, last = pl.pallas_call(
        _main_kernel,
        grid=(_B,),
        in_specs=[
            pl.BlockSpec((1, _T, _NP, _CIN), lambda b: (b, 0, 0, 0)),
            full((_NP, _NP)),
            full((3 * _CIN, _CH)), full((3 * _CIN, _CH)),
            full((3 * _CIN, _CH)), full((3, _CH)),
            full((_CH, _CH)), full((_CH, _CH)), full((_CH, _CH)),
            full((1, _CH)),
            full((3 * _CH, _COUT)), full((3 * _CH, _COUT)),
            full((3 * _CH, _COUT)), full((3, _COUT)),
        ],
        out_specs=[
            pl.BlockSpec((1, 1, _NP), lambda b: (b, 0, 0)),
            pl.BlockSpec((1, 1, _NP), lambda b: (b, 0, 0)),
            pl.BlockSpec((1, _NP, _COUT), lambda b: (b, 0, 0)),
        ],
        out_shape=[
            jax.ShapeDtypeStruct((_B, 1, _NP), f32),
            jax.ShapeDtypeStruct((_B, 1, _NP), f32),
            jax.ShapeDtypeStruct((_B, _NP, _COUT), f32),
        ],
    )(x_pad, lhatT, wp, wq, wr, b1,
      cheb_w[0], cheb_w[1], cheb_w[2], cb, vp, vq, vr, b2)

    gamma_pad = jnp.pad(bn_gamma, (0, _NP - _N)).reshape(_NP, 1)
    beta_pad = jnp.pad(bn_beta, (0, _NP - _N)).reshape(_NP, 1)

    out = pl.pallas_call(
        _final_kernel,
        out_shape=jax.ShapeDtypeStruct((_B, _P + 3), f32),
    )(sums.reshape(_B, _NP).T, sumsq.reshape(_B, _NP).T, last,
      gamma_pad, beta_pad,
      lin_w.T, lin_b.reshape(1, _P), hideout_obs, timestep_obs)

    return out


# speedup vs baseline: 1.0838x; 1.0838x over previous
import functools

import jax
import jax.numpy as jnp
from jax.experimental import pallas as pl
from jax.experimental.pallas import tpu as pltpu
from jax.experimental.pallas import tpu_sc as plsc

_N = 83
_NP = 96
_E = 3403
_B = 32
_T = 50
_CIN = 32
_CH = 16
_COUT = 32
_P = 12
_EPS = 1e-5
_T1 = _T - 2
_T2 = _T - 4
_BN_CNT = _B * _T2 * _COUT


_EP = 3584
_EPT = _EP // 16
_CHUNKS = _EPT // 16


_STRIPE = (_NP * _NP) // 8
_DSLOT = 128


def _lhat_sc_kernel(src_hbm, dst_hbm, out_hbm, src_v, dst_v, degl, degbuf,
                    dinvl, lhatl, accv, tmpv, deg_all, lhat_all):
    cid = jax.lax.axis_index("c")
    sid = jax.lax.axis_index("s")

    @pl.when(cid == 0)
    def _core0():
        base = sid * _EPT
        pltpu.sync_copy(src_hbm.at[pl.ds(base, _EPT)], src_v)
        pltpu.sync_copy(dst_hbm.at[pl.ds(base, _EPT)], dst_v)
        zeros16 = jnp.zeros((16,), jnp.float32)
        lane = jax.lax.iota(jnp.int32, 16)
        for i in range(_DSLOT // 16):
            degl[pl.ds(i * 16, 16)] = zeros16

        for c in range(_CHUNKS):
            sv = src_v[pl.ds(c * 16, 16)]
            dv = dst_v[pl.ds(c * 16, 16)]
            w = jnp.where(sv != dv, 1.0, 0.0).astype(jnp.float32)
            for l in range(16):
                plsc.addupdate_scatter(degl, [sv], w, mask=lane == l)
        pltpu.sync_copy(degl, deg_all.at[pl.ds(sid * _DSLOT, _DSLOT)])
        plsc.subcore_barrier()

        pltpu.sync_copy(deg_all, degbuf)
        for i in range(6):
            d = degbuf[pl.ds(i * 16, 16)]
            for s2 in range(1, 16):
                d = d + degbuf[pl.ds(s2 * _DSLOT + i * 16, 16)]
            s = 0.25 * d + 1.0
            for _ in range(12):
                s = 0.5 * (s + d / s)
            dinvl[pl.ds(i * 16, 16)] = jnp.where(d > 0, 1.0 / s, 0.0)

        for i in range(_NP * _NP // 16):
            lhatl[pl.ds(i * 16, 16)] = zeros16

        for c in range(_CHUNKS):
            sv = src_v[pl.ds(c * 16, 16)]
            dv = dst_v[pl.ds(c * 16, 16)]
            ds_ = plsc.load_gather(dinvl, [sv])
            dd = plsc.load_gather(dinvl, [dv])
            w = jnp.where(sv != dv, 1.0, 0.0).astype(jnp.float32)
            norm = -(ds_ * w * dd)
            fidx = sv * _NP + dv
            for l in range(16):
                plsc.addupdate_scatter(lhatl, [fidx], norm, mask=lane == l)
        nn = _NP * _NP
        pltpu.sync_copy(lhatl, lhat_all.at[pl.ds(sid * nn, nn)])
        plsc.subcore_barrier()

        @pl.when(sid < 8)
        def _():
            pltpu.sync_copy(lhat_all.at[pl.ds(sid * _STRIPE, _STRIPE)], accv)
            for s2 in range(1, 16):
                pltpu.sync_copy(
                    lhat_all.at[pl.ds(s2 * nn + sid * _STRIPE, _STRIPE)], tmpv)
                for k in range(_STRIPE // 16):
                    accv[pl.ds(k * 16, 16)] = (accv[pl.ds(k * 16, 16)]
                                               + tmpv[pl.ds(k * 16, 16)])
            pltpu.sync_copy(accv, out_hbm.at[pl.ds(sid * _STRIPE, _STRIPE)])


def _lhat_call(ei_pad):
    mesh = plsc.VectorSubcoreMesh(core_axis_name="c", subcore_axis_name="s",
                                  num_cores=2, num_subcores=16)
    f32 = jnp.float32
    builder = functools.partial(
        pl.kernel,
        out_type=jax.ShapeDtypeStruct((_NP * _NP,), f32),
        mesh=mesh,
        compiler_params=pltpu.CompilerParams(needs_layout_passes=False),
        scratch_types=[
            pltpu.VMEM((_EPT,), jnp.int32),
            pltpu.VMEM((_EPT,), jnp.int32),
            pltpu.VMEM((_DSLOT,), f32),
            pltpu.VMEM((16 * _DSLOT,), f32),
            pltpu.VMEM((_NP,), f32),
            pltpu.VMEM((_NP * _NP,), f32),
            pltpu.VMEM((_STRIPE,), f32),
            pltpu.VMEM((_STRIPE,), f32),
            pltpu.VMEM_SHARED((16 * _DSLOT,), f32),
            pltpu.VMEM_SHARED((16 * _NP * _NP,), f32),
        ],
    )
    return builder(_lhat_sc_kernel)(ei_pad[0], ei_pad[1]).reshape(_NP, _NP)


def _main_kernel(x_ref, lhatT_ref, wp_ref, wq_ref, wr_ref, b1_ref,
                 cw0_ref, cw1_ref, cw2_ref, cb_ref,
                 vp_ref, vq_ref, vr_ref, b2_ref,
                 sum_ref, sumsq_ref, last_ref):
    x = x_ref[0]
    xp = jnp.concatenate(
        [x, jnp.zeros((_T, _NP - _N, _CIN), jnp.float32)], axis=1)
    xf = xp.reshape(_T * _NP, _CIN)
    x0 = xf[0:_T1 * _NP]
    x1 = xf[_NP:(_T1 + 1) * _NP]
    x2 = xf[2 * _NP:(_T1 + 2) * _NP]
    xc = jnp.concatenate([x0, x1, x2], axis=1)
    p = xc @ wp_ref[...] + b1_ref[0:1]
    q = xc @ wq_ref[...] + b1_ref[1:2]
    r = xc @ wr_ref[...] + b1_ref[2:3]
    f0 = jax.nn.relu(p * jax.nn.sigmoid(q) + r)

    lhatT = lhatT_ref[...]
    zc0 = f0.reshape(_T1, _NP, _CH).transpose(0, 2, 1).reshape(_T1 * _CH, _NP)
    a1 = jnp.dot(zc0, lhatT, preferred_element_type=jnp.float32)
    a2 = 2.0 * jnp.dot(a1, lhatT, preferred_element_type=jnp.float32) - zc0
    f1 = a1.reshape(_T1, _CH, _NP).transpose(0, 2, 1).reshape(_T1 * _NP, _CH)
    f2 = a2.reshape(_T1, _CH, _NP).transpose(0, 2, 1).reshape(_T1 * _NP, _CH)
    out = f0 @ cw0_ref[...] + f1 @ cw1_ref[...] + f2 @ cw2_ref[...] + cb_ref[...]
    tg = jax.nn.relu(out)

    y0 = tg[0:_T2 * _NP]
    y1 = tg[_NP:(_T2 + 1) * _NP]
    y2 = tg[2 * _NP:(_T2 + 2) * _NP]
    yc = jnp.concatenate([y0, y1, y2], axis=1)
    p2 = yc @ vp_ref[...] + b2_ref[0:1]
    q2 = yc @ vq_ref[...] + b2_ref[1:2]
    r2 = yc @ vr_ref[...] + b2_ref[2:3]
    h2 = jax.nn.relu(p2 * jax.nn.sigmoid(q2) + r2)
    h2_3 = h2.reshape(_T2, _NP, _COUT)

    s_tn = jnp.sum(h2_3, axis=2)
    ss_tn = jnp.sum(h2_3 * h2_3, axis=2)
    sum_ref[0] = jnp.sum(s_tn, axis=0, keepdims=True)
    sumsq_ref[0] = jnp.sum(ss_tn, axis=0, keepdims=True)
    last_ref[0] = h2_3[_T2 - 1]


def _final_kernel(sums_ref, sumsq_ref, last_ref, gamma_ref, beta_ref,
                  linw_ref, linb_ref, hide_ref, ts_ref, out_ref):
    s = jnp.sum(sums_ref[...], axis=1, keepdims=True)
    ss = jnp.sum(sumsq_ref[...], axis=1, keepdims=True)
    mean = s / _BN_CNT
    var = ss / _BN_CNT - mean * mean
    inv = gamma_ref[...] * jax.lax.rsqrt(jnp.abs(var) + _EPS)
    shift = beta_ref[...] - mean * inv
    last = last_ref[...]
    h = jax.nn.relu(last * inv[None] + shift[None])
    hl = h.reshape(_B * _NP, _COUT) @ linw_ref[...]
    pooled = jnp.sum(hl.reshape(_B, _NP, _P), axis=1) * (1.0 / _N) + linb_ref[...]
    out_ref[:, 0:_P] = pooled
    out_ref[:, _P:_P + 2] = hide_ref[...]
    out_ref[:, _P + 2:_P + 3] = ts_ref[...]


def kernel(agent_obs, hideout_obs, timestep_obs, num_agents, edge_index,
           tc1_w1, tc1_b1, tc1_w2, tc1_b2, tc1_w3, tc1_b3, cheb_w, cheb_b,
           tc2_w1, tc2_b1, tc2_w2, tc2_b2, tc2_w3, tc2_b3, bn_gamma, bn_beta,
           lin_w, lin_b):
    f32 = jnp.float32

    ei_pad = jnp.pad(edge_index, ((0, 0), (0, _EP - _E)))
    lhatT = _lhat_call(ei_pad)

    def cat_w(w):
        return jnp.concatenate([w[:, :, 0, k].T for k in range(3)], axis=0)

    wp, wq, wr = cat_w(tc1_w1), cat_w(tc1_w2), cat_w(tc1_w3)
    b1 = jnp.stack([tc1_b1, tc1_b2, tc1_b3])
    vp, vq, vr = cat_w(tc2_w1), cat_w(tc2_w2), cat_w(tc2_w3)
    b2 = jnp.stack([tc2_b1, tc2_b2, tc2_b3])
    cb = cheb_b.reshape(1, _CH)

    full = lambda shape: pl.BlockSpec(shape, lambda b: (0,) * len(shape))
    sums, sumsq, last = pl.pallas_call(
        _main_kernel,
        grid=(_B,),
        in_specs=[
            pl.BlockSpec((1, _T, _N, _CIN), lambda b: (b, 0, 0, 0)),
            full((_NP, _NP)),
            full((3 * _CIN, _CH)), full((3 * _CIN, _CH)),
            full((3 * _CIN, _CH)), full((3, _CH)),
            full((_CH, _CH)), full((_CH, _CH)), full((_CH, _CH)),
            full((1, _CH)),
            full((3 * _CH, _COUT)), full((3 * _CH, _COUT)),
            full((3 * _CH, _COUT)), full((3, _COUT)),
        ],
        out_specs=[
            pl.BlockSpec((1, 1, _NP), lambda b: (b, 0, 0)),
            pl.BlockSpec((1, 1, _NP), lambda b: (b, 0, 0)),
            pl.BlockSpec((1, _NP, _COUT), lambda b: (b, 0, 0)),
        ],
        out_shape=[
            jax.ShapeDtypeStruct((_B, 1, _NP), f32),
            jax.ShapeDtypeStruct((_B, 1, _NP), f32),
            jax.ShapeDtypeStruct((_B, _NP, _COUT), f32),
        ],
    )(agent_obs, lhatT, wp, wq, wr, b1,
      cheb_w[0], cheb_w[1], cheb_w[2], cb, vp, vq, vr, b2)

    gamma_pad = jnp.pad(bn_gamma, (0, _NP - _N)).reshape(_NP, 1)
    beta_pad = jnp.pad(bn_beta, (0, _NP - _N)).reshape(_NP, 1)

    out = pl.pallas_call(
        _final_kernel,
        out_shape=jax.ShapeDtypeStruct((_B, _P + 3), f32),
    )(sums.reshape(_B, _NP).T, sumsq.reshape(_B, _NP).T, last,
      gamma_pad, beta_pad,
      lin_w.T, lin_b.reshape(1, _P), hideout_obs, timestep_obs)

    return out
